# baseline (speedup 1.0000x reference)
import functools

import jax
import jax.numpy as jnp
from jax import lax
from jax.experimental import pallas as pl
from jax.experimental.pallas import tpu as pltpu
from jax.experimental.pallas import tpu_sc as plsc

NTYPES = ["drug", "protein", "pathway", "side_effect"]
NNODES = {"drug": 10000, "protein": 50000, "pathway": 10000, "side_effect": 10000}
ETYPES = [
    ("drug", "treats", "side_effect"),
    ("drug", "targets", "protein"),
    ("protein", "in_pathway", "pathway"),
    ("protein", "causes", "side_effect"),
    ("side_effect", "rev_treats", "drug"),
    ("protein", "rev_targets", "drug"),
    ("pathway", "rev_in_pathway", "protein"),
]
D_IN, HID = 128, 256
E = 80000

NC, NS = 2, 16
NW = NC * NS
T = 128
BUF_NT = 24
LOOP_NT = 20
EP = NW * T * BUF_NT
C = 32


def _pad128(n):
    return ((n + 128) // 128) * 128


ZROWS = _pad128(50000) // NS

_MESH = plsc.VectorSubcoreMesh(core_axis_name="c", subcore_axis_name="s")
_SC_PARAMS = pltpu.CompilerParams(use_tc_tiling_on_sc=False)


@functools.lru_cache(maxsize=None)
def _make_agg(n_src, n_dst, nch):
    del n_src
    n_pad = _pad128(n_dst)
    rows_t = n_pad // NS

    def body(*refs):
        tables = refs[:nch]
        src_hbm, dst_hbm, zeros_hbm, out_hbm, acc, src_v, dst_v, rowbuf = refs[nch:]
        cid = lax.axis_index("c")
        sid = lax.axis_index("s")
        base = (cid * NS + sid) * BUF_NT
        pltpu.sync_copy(src_hbm.at[pl.ds(base, BUF_NT)], src_v)
        pltpu.sync_copy(dst_hbm.at[pl.ds(base, BUF_NT)], dst_v)
        for ch in range(nch):
            pltpu.sync_copy(zeros_hbm.at[pl.ds(0, rows_t)],
                            acc.at[pl.ds(sid * rows_t, rows_t)])
            plsc.subcore_barrier()

            def step(j, carry, table=tables[ch]):
                pltpu.sync_copy(table.at[src_v.at[j]], rowbuf)
                pltpu.sync_copy(rowbuf, acc.at[dst_v.at[j]], add=True)
                return carry

            lax.fori_loop(0, LOOP_NT, step, 0)
            plsc.subcore_barrier()
            pltpu.sync_copy(
                acc.at[pl.ds(sid * rows_t, rows_t)],
                out_hbm.at[cid, ch, pl.ds(sid * rows_t, rows_t)])
            plsc.subcore_barrier()

    return pl.kernel(
        body,
        out_type=jax.ShapeDtypeStruct((NC, nch, n_pad, C), jnp.float32),
        mesh=_MESH,
        compiler_params=_SC_PARAMS,
        scratch_types=[
            pltpu.VMEM_SHARED((n_pad, C), jnp.float32),
            pltpu.VMEM((BUF_NT, T), jnp.int32),
            pltpu.VMEM((BUF_NT, T), jnp.int32),
            pltpu.VMEM((T, C), jnp.float32),
        ],
    )


@functools.lru_cache(maxsize=None)
def _make_counts():
    n_dsts = tuple(NNODES[d] for (_, _, d) in ETYPES)

    def body(*refs):
        dsts = refs[:7]
        ones_hbm, zeros_hbm = refs[7:9]
        outs = refs[9:16]
        acc, dst_v, onesbuf = refs[16:]
        cid = lax.axis_index("c")
        sid = lax.axis_index("s")
        base = (cid * NS + sid) * BUF_NT
        pltpu.sync_copy(ones_hbm, onesbuf)
        for t in range(7):
            rows_t = _pad128(n_dsts[t]) // NS
            pltpu.sync_copy(zeros_hbm.at[pl.ds(0, rows_t)],
                            acc.at[pl.ds(sid * rows_t, rows_t)])
            pltpu.sync_copy(dsts[t].at[pl.ds(base, BUF_NT)], dst_v)
            plsc.subcore_barrier()

            def step(j, carry):
                pltpu.sync_copy(onesbuf, acc.at[dst_v.at[j]], add=True)
                return carry

            lax.fori_loop(0, LOOP_NT, step, 0)
            plsc.subcore_barrier()
            pltpu.sync_copy(acc.at[pl.ds(sid * rows_t, rows_t)],
                            outs[t].at[cid, pl.ds(sid * rows_t, rows_t)])
            plsc.subcore_barrier()

    return pl.kernel(
        body,
        out_type=[jax.ShapeDtypeStruct((NC, _pad128(n), 16), jnp.float32)
                  for n in n_dsts],
        mesh=_MESH,
        compiler_params=_SC_PARAMS,
        scratch_types=[
            pltpu.VMEM_SHARED((_pad128(50000), 16), jnp.float32),
            pltpu.VMEM((BUF_NT, T), jnp.int32),
            pltpu.VMEM((T, 16), jnp.float32),
        ],
    )


def _z_body(n_rels, nch, nblk, *refs):
    i = pl.program_id(0)
    aggs = refs[0:n_rels]
    cnts = refs[n_rels:2 * n_rels]
    wls = refs[2 * n_rels:3 * n_rels]
    x_ref, wr_ref, bias_ref, z_ref, st_ref = refs[3 * n_rels:]
    z = jnp.dot(x_ref[...], wr_ref[...], preferred_element_type=jnp.float32)
    z = z + bias_ref[...]
    for r in range(n_rels):
        a = jnp.concatenate(
            [aggs[r][0, c] + aggs[r][1, c] for c in range(nch)], axis=1)
        cnt = cnts[r][0, :, 0:1] + cnts[r][1, :, 0:1]
        mean = a * (1.0 / jnp.maximum(cnt, 1.0))
        z = z + jnp.dot(mean, wls[r][...], preferred_element_type=jnp.float32)
    z_ref[...] = z

    @pl.when(i == 0)
    def _():
        st_ref[...] = jnp.zeros_like(st_ref)

    st_ref[0:1, :] += jnp.sum(z, axis=0, keepdims=True)
    st_ref[1:2, :] += jnp.sum(z * z, axis=0, keepdims=True)


@functools.lru_cache(maxsize=None)
def _make_z(n, d_in, n_rels, blk):
    nblk = n // blk
    nch = d_in // C
    in_specs = (
        [pl.BlockSpec((NC, nch, blk, C), lambda i: (0, 0, i, 0))
         for _ in range(n_rels)]
        + [pl.BlockSpec((NC, blk, 16), lambda i: (0, i, 0)) for _ in range(n_rels)]
        + [pl.BlockSpec((d_in, HID), lambda i: (0, 0)) for _ in range(n_rels)]
        + [
            pl.BlockSpec((blk, d_in), lambda i: (i, 0)),
            pl.BlockSpec((d_in, HID), lambda i: (0, 0)),
            pl.BlockSpec((1, HID), lambda i: (0, 0)),
        ]
    )
    return pl.pallas_call(
        functools.partial(_z_body, n_rels, nch, nblk),
        grid=(nblk,),
        in_specs=in_specs,
        out_specs=[
            pl.BlockSpec((blk, HID), lambda i: (i, 0)),
            pl.BlockSpec((8, HID), lambda i: (0, 0)),
        ],
        out_shape=[
            jax.ShapeDtypeStruct((n, HID), jnp.float32),
            jax.ShapeDtypeStruct((8, HID), jnp.float32),
        ],
    )


def _bn_body(n, nch_out, z_ref, st_ref, g_ref, b_ref, *out_refs):
    inv_n = 1.0 / n
    m = st_ref[0:1, :] * inv_n
    var = st_ref[1:2, :] * inv_n - m * m
    scale = lax.rsqrt(var + 1e-5) * g_ref[...]
    y = jnp.maximum((z_ref[...] - m) * scale + b_ref[...], 0.0)
    out_refs[0][...] = y
    for c in range(nch_out):
        out_refs[1 + c][...] = y[:, c * C:(c + 1) * C]


@functools.lru_cache(maxsize=None)
def _make_bn(n, nch_out, blk):
    nblk = n // blk
    out_specs = [pl.BlockSpec((blk, HID), lambda i: (i, 0))]
    out_shape = [jax.ShapeDtypeStruct((n, HID), jnp.float32)]
    for _ in range(nch_out):
        out_specs.append(pl.BlockSpec((blk, C), lambda i: (i, 0)))
        out_shape.append(jax.ShapeDtypeStruct((n, C), jnp.float32))
    return pl.pallas_call(
        functools.partial(_bn_body, n, nch_out),
        grid=(nblk,),
        in_specs=[
            pl.BlockSpec((blk, HID), lambda i: (i, 0)),
            pl.BlockSpec((8, HID), lambda i: (0, 0)),
            pl.BlockSpec((1, HID), lambda i: (0, 0)),
            pl.BlockSpec((1, HID), lambda i: (0, 0)),
        ],
        out_specs=out_specs,
        out_shape=out_shape,
    )


def _pad_edges(e, n_dst):
    e = e.astype(jnp.int32)

    def lay(v, fill):
        v = jnp.concatenate(
            [v, jnp.full((NW * LOOP_NT * T - E,), fill, jnp.int32)])
        v = v.reshape(NW, LOOP_NT, T)
        pad = jnp.full((NW, BUF_NT - LOOP_NT, T), fill, jnp.int32)
        return jnp.concatenate([v, pad], axis=1).reshape(EP // T, T)

    return lay(e[0], 0), lay(e[1], n_dst)


def _layer(xd, x_chunks, edges, counts, params, layer, blk=1000):
    nch = xd[NTYPES[0]].shape[1] // C
    aggs = {}
    for (s, rel, d) in ETYPES:
        src2d, dst2d, zeros32 = edges[rel]
        aggs[rel] = _make_agg(NNODES[s], NNODES[d], nch)(
            *x_chunks[s], src2d, dst2d, zeros32)
    out = {}
    in_rels = {nt: [] for nt in NTYPES}
    for (s, rel, d) in ETYPES:
        in_rels[d].append(rel)
    for nt in NTYPES:
        rels = in_rels[nt]
        n = NNODES[nt]
        d_in = xd[nt].shape[1]
        wl_list = [params["W%dl_%s" % (layer, r)].T for r in rels]
        wr = sum(params["W%dr_%s" % (layer, r)] for r in rels).T
        bias = sum(params["b%dl_%s" % (layer, r)] for r in rels).reshape(1, HID)
        z, st = _make_z(n, d_in, len(rels), blk)(
            *[aggs[r] for r in rels], *[counts[r] for r in rels],
            *wl_list, xd[nt], wr, bias)
        out[nt] = (z, st)
    return out


def kernel(x_drug, x_protein, x_pathway, x_side_effect, ei_treats, ei_targets, ei_in_pathway, ei_causes, ei_rev_treats, ei_rev_targets, ei_rev_in_pathway, W1l_treats, b1l_treats, W1r_treats, W2l_treats, b2l_treats, W2r_treats, W1l_targets, b1l_targets, W1r_targets, W2l_targets, b2l_targets, W2r_targets, W1l_in_pathway, b1l_in_pathway, W1r_in_pathway, W2l_in_pathway, b2l_in_pathway, W2r_in_pathway, W1l_causes, b1l_causes, W1r_causes, W2l_causes, b2l_causes, W2r_causes, W1l_rev_treats, b1l_rev_treats, W1r_rev_treats, W2l_rev_treats, b2l_rev_treats, W2r_rev_treats, W1l_rev_targets, b1l_rev_targets, W1r_rev_targets, W2l_rev_targets, b2l_rev_targets, W2r_rev_targets, W1l_rev_in_pathway, b1l_rev_in_pathway, W1r_rev_in_pathway, W2l_rev_in_pathway, b2l_rev_in_pathway, W2r_rev_in_pathway, bn1_g, bn1_b, bn2_g, bn2_b):
    params = dict(locals())
    xd = {nt: params["x_" + nt] for nt in NTYPES}

    zeros32 = jnp.zeros((ZROWS, C), jnp.float32)
    zeros16 = jnp.zeros((ZROWS, 16), jnp.float32)
    ones16 = jnp.ones((T, 16), jnp.float32)
    edges = {}
    for (s, rel, d) in ETYPES:
        src2d, dst2d = _pad_edges(params["ei_" + rel], NNODES[d])
        edges[rel] = (src2d, dst2d, zeros32)

    cnt_list = _make_counts()(*[edges[rel][1] for (_, rel, _) in ETYPES],
                              ones16, zeros16)
    counts = {rel: cnt_list[t] for t, (_, rel, _) in enumerate(ETYPES)}

    x_chunks = {nt: [xd[nt][:, c * C:(c + 1) * C] for c in range(D_IN // C)]
                for nt in NTYPES}
    z1 = _layer(xd, x_chunks, edges, counts, params, 1)
    x1, x1_chunks = {}, {}
    for nt in NTYPES:
        z, st = z1[nt]
        outs = _make_bn(NNODES[nt], HID // C, 1000)(
            z, st, bn1_g.reshape(1, HID), bn1_b.reshape(1, HID))
        x1[nt] = outs[0]
        x1_chunks[nt] = outs[1:]

    z2 = _layer(x1, x1_chunks, edges, counts, params, 2)
    res = []
    for nt in NTYPES:
        z, st = z2[nt]
        outs = _make_bn(NNODES[nt], 0, 1000)(
            z, st, bn2_g.reshape(1, HID), bn2_b.reshape(1, HID))
        res.append(outs[0])
    return tuple(res)


# speedup vs baseline: 1.1775x; 1.1775x over previous
import functools

import jax
import jax.numpy as jnp
from jax import lax
from jax.experimental import pallas as pl
from jax.experimental.pallas import tpu as pltpu
from jax.experimental.pallas import tpu_sc as plsc

NTYPES = ["drug", "protein", "pathway", "side_effect"]
NNODES = {"drug": 10000, "protein": 50000, "pathway": 10000, "side_effect": 10000}
ETYPES = [
    ("drug", "treats", "side_effect"),
    ("drug", "targets", "protein"),
    ("protein", "in_pathway", "pathway"),
    ("protein", "causes", "side_effect"),
    ("side_effect", "rev_treats", "drug"),
    ("protein", "rev_targets", "drug"),
    ("pathway", "rev_in_pathway", "protein"),
]
D_IN, HID = 128, 256
E = 80000

NC, NS = 2, 16
NW = NC * NS
T = 128
BUF_NT = 24
LOOP_NT = 20
EP = NW * T * BUF_NT
C = 32


def _pad128(n):
    return ((n + 128) // 128) * 128


ZROWS = _pad128(50000) // NS

_MESH = plsc.VectorSubcoreMesh(core_axis_name="c", subcore_axis_name="s")
_SC_PARAMS = pltpu.CompilerParams(use_tc_tiling_on_sc=False)


@functools.lru_cache(maxsize=None)
def _make_agg(n_src, n_dst, nch):
    del n_src
    n_pad = _pad128(n_dst)
    rows_t = n_pad // NS
    NBUF = 4

    def body(*refs):
        tables = refs[:nch]
        src_hbm, dst_hbm, zeros_hbm, out_hbm = refs[nch:nch + 4]
        acc, src_v, dst_v, rowbufs = refs[nch + 4:nch + 8]
        gsems = refs[nch + 8:]
        cid = lax.axis_index("c")
        sid = lax.axis_index("s")
        base = (cid * NS + sid) * BUF_NT
        pltpu.sync_copy(src_hbm.at[pl.ds(base, BUF_NT)], src_v)
        pltpu.sync_copy(dst_hbm.at[pl.ds(base, BUF_NT)], dst_v)

        for ch in range(nch):
            table = tables[ch]
            pltpu.sync_copy(zeros_hbm.at[pl.ds(0, rows_t)],
                            acc.at[pl.ds(sid * rows_t, rows_t)])
            plsc.subcore_barrier()

            for b in range(NBUF):
                pltpu.async_copy(table.at[src_v.at[b]], rowbufs.at[b], gsems[b])

            def outer(g, carry):
                for b in range(NBUF):
                    j = g * NBUF + b
                    pltpu.make_async_copy(
                        table.at[src_v.at[j]], rowbufs.at[b], gsems[b]).wait()
                    pltpu.sync_copy(rowbufs.at[b], acc.at[dst_v.at[j]],
                                    add=True)

                    @pl.when(j + NBUF < LOOP_NT)
                    def _():
                        pltpu.async_copy(table.at[src_v.at[j + NBUF]],
                                         rowbufs.at[b], gsems[b])
                return carry

            lax.fori_loop(0, LOOP_NT // NBUF, outer, 0)
            plsc.subcore_barrier()
            pltpu.sync_copy(
                acc.at[pl.ds(sid * rows_t, rows_t)],
                out_hbm.at[cid, ch, pl.ds(sid * rows_t, rows_t)])
            plsc.subcore_barrier()

    return pl.kernel(
        body,
        out_type=jax.ShapeDtypeStruct((NC, nch, n_pad, C), jnp.float32),
        mesh=_MESH,
        compiler_params=_SC_PARAMS,
        scratch_types=[
            pltpu.VMEM_SHARED((n_pad, C), jnp.float32),
            pltpu.VMEM((BUF_NT, T), jnp.int32),
            pltpu.VMEM((BUF_NT, T), jnp.int32),
            pltpu.VMEM((4, T, C), jnp.float32),
        ] + [pltpu.SemaphoreType.DMA] * 4,
    )


@functools.lru_cache(maxsize=None)
def _make_counts():
    n_dsts = tuple(NNODES[d] for (_, _, d) in ETYPES)

    def body(*refs):
        dsts = refs[:7]
        ones_hbm, zeros_hbm = refs[7:9]
        outs = refs[9:16]
        acc, dst_v, onesbuf = refs[16:]
        cid = lax.axis_index("c")
        sid = lax.axis_index("s")
        base = (cid * NS + sid) * BUF_NT
        pltpu.sync_copy(ones_hbm, onesbuf)
        for t in range(7):
            rows_t = _pad128(n_dsts[t]) // NS
            pltpu.sync_copy(zeros_hbm.at[pl.ds(0, rows_t)],
                            acc.at[pl.ds(sid * rows_t, rows_t)])
            pltpu.sync_copy(dsts[t].at[pl.ds(base, BUF_NT)], dst_v)
            plsc.subcore_barrier()

            def step(j, carry):
                pltpu.sync_copy(onesbuf, acc.at[dst_v.at[j]], add=True)
                return carry

            lax.fori_loop(0, LOOP_NT, step, 0)
            plsc.subcore_barrier()
            pltpu.sync_copy(acc.at[pl.ds(sid * rows_t, rows_t)],
                            outs[t].at[cid, pl.ds(sid * rows_t, rows_t)])
            plsc.subcore_barrier()

    return pl.kernel(
        body,
        out_type=[jax.ShapeDtypeStruct((NC, _pad128(n), 16), jnp.float32)
                  for n in n_dsts],
        mesh=_MESH,
        compiler_params=_SC_PARAMS,
        scratch_types=[
            pltpu.VMEM_SHARED((_pad128(50000), 16), jnp.float32),
            pltpu.VMEM((BUF_NT, T), jnp.int32),
            pltpu.VMEM((T, 16), jnp.float32),
        ],
    )


def _z_body(n_rels, nch, nblk, *refs):
    i = pl.program_id(0)
    aggs = refs[0:n_rels]
    cnts = refs[n_rels:2 * n_rels]
    wls = refs[2 * n_rels:3 * n_rels]
    x_ref, wr_ref, bias_ref, z_ref, st_ref = refs[3 * n_rels:]
    z = jnp.dot(x_ref[...], wr_ref[...], preferred_element_type=jnp.float32)
    z = z + bias_ref[...]
    for r in range(n_rels):
        a = jnp.concatenate(
            [aggs[r][0, c] + aggs[r][1, c] for c in range(nch)], axis=1)
        cnt = cnts[r][0, :, 0:1] + cnts[r][1, :, 0:1]
        mean = a * (1.0 / jnp.maximum(cnt, 1.0))
        z = z + jnp.dot(mean, wls[r][...], preferred_element_type=jnp.float32)
    z_ref[...] = z

    @pl.when(i == 0)
    def _():
        st_ref[...] = jnp.zeros_like(st_ref)

    st_ref[0:1, :] += jnp.sum(z, axis=0, keepdims=True)
    st_ref[1:2, :] += jnp.sum(z * z, axis=0, keepdims=True)


@functools.lru_cache(maxsize=None)
def _make_z(n, d_in, n_rels, blk):
    nblk = n // blk
    nch = d_in // C
    in_specs = (
        [pl.BlockSpec((NC, nch, blk, C), lambda i: (0, 0, i, 0))
         for _ in range(n_rels)]
        + [pl.BlockSpec((NC, blk, 16), lambda i: (0, i, 0)) for _ in range(n_rels)]
        + [pl.BlockSpec((d_in, HID), lambda i: (0, 0)) for _ in range(n_rels)]
        + [
            pl.BlockSpec((blk, d_in), lambda i: (i, 0)),
            pl.BlockSpec((d_in, HID), lambda i: (0, 0)),
            pl.BlockSpec((1, HID), lambda i: (0, 0)),
        ]
    )
    return pl.pallas_call(
        functools.partial(_z_body, n_rels, nch, nblk),
        grid=(nblk,),
        in_specs=in_specs,
        out_specs=[
            pl.BlockSpec((blk, HID), lambda i: (i, 0)),
            pl.BlockSpec((8, HID), lambda i: (0, 0)),
        ],
        out_shape=[
            jax.ShapeDtypeStruct((n, HID), jnp.float32),
            jax.ShapeDtypeStruct((8, HID), jnp.float32),
        ],
    )


def _bn_body(n, nch_out, z_ref, st_ref, g_ref, b_ref, *out_refs):
    inv_n = 1.0 / n
    m = st_ref[0:1, :] * inv_n
    var = st_ref[1:2, :] * inv_n - m * m
    scale = lax.rsqrt(var + 1e-5) * g_ref[...]
    y = jnp.maximum((z_ref[...] - m) * scale + b_ref[...], 0.0)
    out_refs[0][...] = y
    for c in range(nch_out):
        out_refs[1 + c][...] = y[:, c * C:(c + 1) * C]


@functools.lru_cache(maxsize=None)
def _make_bn(n, nch_out, blk):
    nblk = n // blk
    out_specs = [pl.BlockSpec((blk, HID), lambda i: (i, 0))]
    out_shape = [jax.ShapeDtypeStruct((n, HID), jnp.float32)]
    for _ in range(nch_out):
        out_specs.append(pl.BlockSpec((blk, C), lambda i: (i, 0)))
        out_shape.append(jax.ShapeDtypeStruct((n, C), jnp.float32))
    return pl.pallas_call(
        functools.partial(_bn_body, n, nch_out),
        grid=(nblk,),
        in_specs=[
            pl.BlockSpec((blk, HID), lambda i: (i, 0)),
            pl.BlockSpec((8, HID), lambda i: (0, 0)),
            pl.BlockSpec((1, HID), lambda i: (0, 0)),
            pl.BlockSpec((1, HID), lambda i: (0, 0)),
        ],
        out_specs=out_specs,
        out_shape=out_shape,
    )


def _pad_edges(e, n_dst):
    e = e.astype(jnp.int32)

    def lay(v, fill):
        v = jnp.concatenate(
            [v, jnp.full((NW * LOOP_NT * T - E,), fill, jnp.int32)])
        v = v.reshape(NW, LOOP_NT, T)
        pad = jnp.full((NW, BUF_NT - LOOP_NT, T), fill, jnp.int32)
        return jnp.concatenate([v, pad], axis=1).reshape(EP // T, T)

    return lay(e[0], 0), lay(e[1], n_dst)


def _layer(xd, x_chunks, edges, counts, params, layer, blk=1000):
    nch = xd[NTYPES[0]].shape[1] // C
    aggs = {}
    for (s, rel, d) in ETYPES:
        src2d, dst2d, zeros32 = edges[rel]
        aggs[rel] = _make_agg(NNODES[s], NNODES[d], nch)(
            *x_chunks[s], src2d, dst2d, zeros32)
    out = {}
    in_rels = {nt: [] for nt in NTYPES}
    for (s, rel, d) in ETYPES:
        in_rels[d].append(rel)
    for nt in NTYPES:
        rels = in_rels[nt]
        n = NNODES[nt]
        d_in = xd[nt].shape[1]
        wl_list = [params["W%dl_%s" % (layer, r)].T for r in rels]
        wr = sum(params["W%dr_%s" % (layer, r)] for r in rels).T
        bias = sum(params["b%dl_%s" % (layer, r)] for r in rels).reshape(1, HID)
        z, st = _make_z(n, d_in, len(rels), blk)(
            *[aggs[r] for r in rels], *[counts[r] for r in rels],
            *wl_list, xd[nt], wr, bias)
        out[nt] = (z, st)
    return out


def kernel(x_drug, x_protein, x_pathway, x_side_effect, ei_treats, ei_targets, ei_in_pathway, ei_causes, ei_rev_treats, ei_rev_targets, ei_rev_in_pathway, W1l_treats, b1l_treats, W1r_treats, W2l_treats, b2l_treats, W2r_treats, W1l_targets, b1l_targets, W1r_targets, W2l_targets, b2l_targets, W2r_targets, W1l_in_pathway, b1l_in_pathway, W1r_in_pathway, W2l_in_pathway, b2l_in_pathway, W2r_in_pathway, W1l_causes, b1l_causes, W1r_causes, W2l_causes, b2l_causes, W2r_causes, W1l_rev_treats, b1l_rev_treats, W1r_rev_treats, W2l_rev_treats, b2l_rev_treats, W2r_rev_treats, W1l_rev_targets, b1l_rev_targets, W1r_rev_targets, W2l_rev_targets, b2l_rev_targets, W2r_rev_targets, W1l_rev_in_pathway, b1l_rev_in_pathway, W1r_rev_in_pathway, W2l_rev_in_pathway, b2l_rev_in_pathway, W2r_rev_in_pathway, bn1_g, bn1_b, bn2_g, bn2_b):
    params = dict(locals())
    xd = {nt: params["x_" + nt] for nt in NTYPES}

    zeros32 = jnp.zeros((ZROWS, C), jnp.float32)
    zeros16 = jnp.zeros((ZROWS, 16), jnp.float32)
    ones16 = jnp.ones((T, 16), jnp.float32)
    edges = {}
    for (s, rel, d) in ETYPES:
        src2d, dst2d = _pad_edges(params["ei_" + rel], NNODES[d])
        edges[rel] = (src2d, dst2d, zeros32)

    cnt_list = _make_counts()(*[edges[rel][1] for (_, rel, _) in ETYPES],
                              ones16, zeros16)
    counts = {rel: cnt_list[t] for t, (_, rel, _) in enumerate(ETYPES)}

    x_chunks = {nt: [xd[nt][:, c * C:(c + 1) * C] for c in range(D_IN // C)]
                for nt in NTYPES}
    z1 = _layer(xd, x_chunks, edges, counts, params, 1)
    x1, x1_chunks = {}, {}
    for nt in NTYPES:
        z, st = z1[nt]
        outs = _make_bn(NNODES[nt], HID // C, 1000)(
            z, st, bn1_g.reshape(1, HID), bn1_b.reshape(1, HID))
        x1[nt] = outs[0]
        x1_chunks[nt] = outs[1:]

    z2 = _layer(x1, x1_chunks, edges, counts, params, 2)
    res = []
    for nt in NTYPES:
        z, st = z2[nt]
        outs = _make_bn(NNODES[nt], 0, 1000)(
            z, st, bn2_g.reshape(1, HID), bn2_b.reshape(1, HID))
        res.append(outs[0])
    return tuple(res)


# speedup vs baseline: 1.3105x; 1.1129x over previous
import functools

import jax
import jax.numpy as jnp
from jax import lax
from jax.experimental import pallas as pl
from jax.experimental.pallas import tpu as pltpu
from jax.experimental.pallas import tpu_sc as plsc

NTYPES = ["drug", "protein", "pathway", "side_effect"]
NNODES = {"drug": 10000, "protein": 50000, "pathway": 10000, "side_effect": 10000}
ETYPES = [
    ("drug", "treats", "side_effect"),
    ("drug", "targets", "protein"),
    ("protein", "in_pathway", "pathway"),
    ("protein", "causes", "side_effect"),
    ("side_effect", "rev_treats", "drug"),
    ("protein", "rev_targets", "drug"),
    ("pathway", "rev_in_pathway", "protein"),
]
D_IN, HID = 128, 256
E = 80000

NS = 16
T = 128
NT = 40
EP = NS * T * NT
C = 32


def _pad128(n):
    return ((n + 128) // 128) * 128


ZROWS = _pad128(50000) // NS

_MESH = plsc.VectorSubcoreMesh(core_axis_name="c", subcore_axis_name="s",
                               num_cores=1)
_SC_PARAMS = pltpu.CompilerParams(use_tc_tiling_on_sc=False)


@functools.lru_cache(maxsize=None)
def _make_agg(n_src, n_dst, nch):
    del n_src
    n_pad = _pad128(n_dst)
    rows_t = n_pad // NS
    NBUF = 4

    def body(*refs):
        tables = refs[:nch]
        src_hbm, dst_hbm, zeros_hbm, out_hbm = refs[nch:nch + 4]
        acc, src_v, dst_v, rowbufs = refs[nch + 4:nch + 8]
        gsems = refs[nch + 8:]
        sid = lax.axis_index("s")
        base = sid * NT
        pltpu.sync_copy(src_hbm.at[pl.ds(base, NT)], src_v)
        pltpu.sync_copy(dst_hbm.at[pl.ds(base, NT)], dst_v)

        for ch in range(nch):
            table = tables[ch]
            pltpu.sync_copy(zeros_hbm.at[pl.ds(0, rows_t)],
                            acc.at[pl.ds(sid * rows_t, rows_t)])
            plsc.subcore_barrier()

            for b in range(NBUF):
                pltpu.async_copy(table.at[src_v.at[b]], rowbufs.at[b], gsems[b])

            def outer(g, carry):
                for b in range(NBUF):
                    j = g * NBUF + b
                    pltpu.make_async_copy(
                        table.at[src_v.at[j]], rowbufs.at[b], gsems[b]).wait()
                    pltpu.sync_copy(rowbufs.at[b], acc.at[dst_v.at[j]],
                                    add=True)

                    @pl.when(j + NBUF < NT)
                    def _():
                        pltpu.async_copy(table.at[src_v.at[j + NBUF]],
                                         rowbufs.at[b], gsems[b])
                return carry

            lax.fori_loop(0, NT // NBUF, outer, 0)
            plsc.subcore_barrier()
            pltpu.sync_copy(
                acc.at[pl.ds(sid * rows_t, rows_t)],
                out_hbm.at[ch, pl.ds(sid * rows_t, rows_t)])
            plsc.subcore_barrier()

    return pl.kernel(
        body,
        out_type=jax.ShapeDtypeStruct((nch, n_pad, C), jnp.float32),
        mesh=_MESH,
        compiler_params=_SC_PARAMS,
        scratch_types=[
            pltpu.VMEM_SHARED((n_pad, C), jnp.float32),
            pltpu.VMEM((NT, T), jnp.int32),
            pltpu.VMEM((NT, T), jnp.int32),
            pltpu.VMEM((4, T, C), jnp.float32),
        ] + [pltpu.SemaphoreType.DMA] * 4,
    )


@functools.lru_cache(maxsize=None)
def _make_counts():
    n_dsts = tuple(NNODES[d] for (_, _, d) in ETYPES)

    def body(*refs):
        dsts = refs[:7]
        ones_hbm, zeros_hbm = refs[7:9]
        outs = refs[9:16]
        acc, dst_v, onesbuf = refs[16:]
        sid = lax.axis_index("s")
        base = sid * NT
        pltpu.sync_copy(ones_hbm, onesbuf)
        for t in range(7):
            rows_t = _pad128(n_dsts[t]) // NS
            pltpu.sync_copy(zeros_hbm.at[pl.ds(0, rows_t)],
                            acc.at[pl.ds(sid * rows_t, rows_t)])
            pltpu.sync_copy(dsts[t].at[pl.ds(base, NT)], dst_v)
            plsc.subcore_barrier()

            def step(j, carry):
                pltpu.sync_copy(onesbuf, acc.at[dst_v.at[j]], add=True)
                return carry

            lax.fori_loop(0, NT, step, 0)
            plsc.subcore_barrier()
            pltpu.sync_copy(acc.at[pl.ds(sid * rows_t, rows_t)],
                            outs[t].at[pl.ds(sid * rows_t, rows_t)])
            plsc.subcore_barrier()

    return pl.kernel(
        body,
        out_type=[jax.ShapeDtypeStruct((_pad128(n), 16), jnp.float32)
                  for n in n_dsts],
        mesh=_MESH,
        compiler_params=_SC_PARAMS,
        scratch_types=[
            pltpu.VMEM_SHARED((_pad128(50000), 16), jnp.float32),
            pltpu.VMEM((NT, T), jnp.int32),
            pltpu.VMEM((T, 16), jnp.float32),
        ],
    )


def _z_body(n_rels, nch, nblk, *refs):
    i = pl.program_id(0)
    aggs = refs[0:n_rels]
    cnts = refs[n_rels:2 * n_rels]
    wls = refs[2 * n_rels:3 * n_rels]
    x_ref, wr_ref, bias_ref, z_ref, st_ref = refs[3 * n_rels:]
    z = jnp.dot(x_ref[...], wr_ref[...], preferred_element_type=jnp.float32)
    z = z + bias_ref[...]
    for r in range(n_rels):
        a = jnp.concatenate([aggs[r][c] for c in range(nch)], axis=1)
        cnt = cnts[r][:, 0:1]
        mean = a * (1.0 / jnp.maximum(cnt, 1.0))
        z = z + jnp.dot(mean, wls[r][...], preferred_element_type=jnp.float32)
    z_ref[...] = z

    @pl.when(i == 0)
    def _():
        st_ref[...] = jnp.zeros_like(st_ref)

    st_ref[0:1, :] += jnp.sum(z, axis=0, keepdims=True)
    st_ref[1:2, :] += jnp.sum(z * z, axis=0, keepdims=True)


@functools.lru_cache(maxsize=None)
def _make_z(n, d_in, n_rels, blk):
    nblk = n // blk
    nch = d_in // C
    in_specs = (
        [pl.BlockSpec((nch, blk, C), lambda i: (0, i, 0))
         for _ in range(n_rels)]
        + [pl.BlockSpec((blk, 16), lambda i: (i, 0)) for _ in range(n_rels)]
        + [pl.BlockSpec((d_in, HID), lambda i: (0, 0)) for _ in range(n_rels)]
        + [
            pl.BlockSpec((blk, d_in), lambda i: (i, 0)),
            pl.BlockSpec((d_in, HID), lambda i: (0, 0)),
            pl.BlockSpec((1, HID), lambda i: (0, 0)),
        ]
    )
    return pl.pallas_call(
        functools.partial(_z_body, n_rels, nch, nblk),
        grid=(nblk,),
        in_specs=in_specs,
        out_specs=[
            pl.BlockSpec((blk, HID), lambda i: (i, 0)),
            pl.BlockSpec((8, HID), lambda i: (0, 0)),
        ],
        out_shape=[
            jax.ShapeDtypeStruct((n, HID), jnp.float32),
            jax.ShapeDtypeStruct((8, HID), jnp.float32),
        ],
    )


def _bn_body(n, nch_out, z_ref, st_ref, g_ref, b_ref, *out_refs):
    inv_n = 1.0 / n
    m = st_ref[0:1, :] * inv_n
    var = st_ref[1:2, :] * inv_n - m * m
    scale = lax.rsqrt(var + 1e-5) * g_ref[...]
    y = jnp.maximum((z_ref[...] - m) * scale + b_ref[...], 0.0)
    out_refs[0][...] = y
    for c in range(nch_out):
        out_refs[1 + c][...] = y[:, c * C:(c + 1) * C]


@functools.lru_cache(maxsize=None)
def _make_bn(n, nch_out, blk):
    nblk = n // blk
    out_specs = [pl.BlockSpec((blk, HID), lambda i: (i, 0))]
    out_shape = [jax.ShapeDtypeStruct((n, HID), jnp.float32)]
    for _ in range(nch_out):
        out_specs.append(pl.BlockSpec((blk, C), lambda i: (i, 0)))
        out_shape.append(jax.ShapeDtypeStruct((n, C), jnp.float32))
    return pl.pallas_call(
        functools.partial(_bn_body, n, nch_out),
        grid=(nblk,),
        in_specs=[
            pl.BlockSpec((blk, HID), lambda i: (i, 0)),
            pl.BlockSpec((8, HID), lambda i: (0, 0)),
            pl.BlockSpec((1, HID), lambda i: (0, 0)),
            pl.BlockSpec((1, HID), lambda i: (0, 0)),
        ],
        out_specs=out_specs,
        out_shape=out_shape,
    )


def _pad_edges(e, n_dst):
    e = e.astype(jnp.int32)
    src = jnp.concatenate([e[0], jnp.zeros((EP - E,), jnp.int32)])
    dst = jnp.concatenate([e[1], jnp.full((EP - E,), n_dst, jnp.int32)])
    return src.reshape(EP // T, T), dst.reshape(EP // T, T)


def _layer(xd, x_chunks, edges, counts, params, layer, blk=1000):
    nch = xd[NTYPES[0]].shape[1] // C
    aggs = {}
    for (s, rel, d) in ETYPES:
        src2d, dst2d, zeros32 = edges[rel]
        aggs[rel] = _make_agg(NNODES[s], NNODES[d], nch)(
            *x_chunks[s], src2d, dst2d, zeros32)
    out = {}
    in_rels = {nt: [] for nt in NTYPES}
    for (s, rel, d) in ETYPES:
        in_rels[d].append(rel)
    for nt in NTYPES:
        rels = in_rels[nt]
        n = NNODES[nt]
        d_in = xd[nt].shape[1]
        wl_list = [params["W%dl_%s" % (layer, r)].T for r in rels]
        wr = sum(params["W%dr_%s" % (layer, r)] for r in rels).T
        bias = sum(params["b%dl_%s" % (layer, r)] for r in rels).reshape(1, HID)
        z, st = _make_z(n, d_in, len(rels), blk)(
            *[aggs[r] for r in rels], *[counts[r] for r in rels],
            *wl_list, xd[nt], wr, bias)
        out[nt] = (z, st)
    return out


def kernel(x_drug, x_protein, x_pathway, x_side_effect, ei_treats, ei_targets, ei_in_pathway, ei_causes, ei_rev_treats, ei_rev_targets, ei_rev_in_pathway, W1l_treats, b1l_treats, W1r_treats, W2l_treats, b2l_treats, W2r_treats, W1l_targets, b1l_targets, W1r_targets, W2l_targets, b2l_targets, W2r_targets, W1l_in_pathway, b1l_in_pathway, W1r_in_pathway, W2l_in_pathway, b2l_in_pathway, W2r_in_pathway, W1l_causes, b1l_causes, W1r_causes, W2l_causes, b2l_causes, W2r_causes, W1l_rev_treats, b1l_rev_treats, W1r_rev_treats, W2l_rev_treats, b2l_rev_treats, W2r_rev_treats, W1l_rev_targets, b1l_rev_targets, W1r_rev_targets, W2l_rev_targets, b2l_rev_targets, W2r_rev_targets, W1l_rev_in_pathway, b1l_rev_in_pathway, W1r_rev_in_pathway, W2l_rev_in_pathway, b2l_rev_in_pathway, W2r_rev_in_pathway, bn1_g, bn1_b, bn2_g, bn2_b):
    params = dict(locals())
    xd = {nt: params["x_" + nt] for nt in NTYPES}

    zeros32 = jnp.zeros((ZROWS, C), jnp.float32)
    zeros16 = jnp.zeros((ZROWS, 16), jnp.float32)
    ones16 = jnp.ones((T, 16), jnp.float32)
    edges = {}
    for (s, rel, d) in ETYPES:
        src2d, dst2d = _pad_edges(params["ei_" + rel], NNODES[d])
        edges[rel] = (src2d, dst2d, zeros32)

    cnt_list = _make_counts()(*[edges[rel][1] for (_, rel, _) in ETYPES],
                              ones16, zeros16)
    counts = {rel: cnt_list[t] for t, (_, rel, _) in enumerate(ETYPES)}

    x_chunks = {nt: [xd[nt][:, c * C:(c + 1) * C] for c in range(D_IN // C)]
                for nt in NTYPES}
    z1 = _layer(xd, x_chunks, edges, counts, params, 1)
    x1, x1_chunks = {}, {}
    for nt in NTYPES:
        z, st = z1[nt]
        outs = _make_bn(NNODES[nt], HID // C, 1000)(
            z, st, bn1_g.reshape(1, HID), bn1_b.reshape(1, HID))
        x1[nt] = outs[0]
        x1_chunks[nt] = outs[1:]

    z2 = _layer(x1, x1_chunks, edges, counts, params, 2)
    res = []
    for nt in NTYPES:
        z, st = z2[nt]
        outs = _make_bn(NNODES[nt], 0, 1000)(
            z, st, bn2_g.reshape(1, HID), bn2_b.reshape(1, HID))
        res.append(outs[0])
    return tuple(res)
